# baseline (speedup 1.0000x reference)
import functools

import jax
import jax.numpy as jnp
from jax import lax
from jax.experimental import pallas as pl
from jax.experimental.pallas import tpu as pltpu
from jax.experimental.pallas import tpu_sc as plsc

NUM_IDS = 4096 * 200
EMBED = 64
G = 4
ROWW = EMBED * G
NGRP = NUM_IDS // G
NC, NS = 2, 16
NW = NC * NS
PER_W = NGRP // NW
BLK = 128
SLOTS = PER_W // BLK
RING = 2
L = 16


def _sc_body(ids_hbm, table_hbm, out_hbm, ids_v, idx_v, rows_v,
             gsem, ssem):
    sid = lax.axis_index("s")
    wid = sid * NC + lax.axis_index("c")
    base_w = wid * PER_W

    pltpu.sync_copy(ids_hbm.at[wid], ids_v)

    tbase = wid * 81

    def compute(k, carry):
        w = ids_v[pl.ds(k * L, L)]
        va = w & 255
        vb = (w >> 8) & 255
        vc = (w >> 16) & 255
        vd = (w >> 24) & 255
        idx_v[pl.ds(k * L, L)] = tbase + ((va * 3 + vb) * 3 + vc) * 3 + vd
        return carry

    lax.fori_loop(0, PER_W // L, compute, 0)

    def gather(s, b):
        return pltpu.make_async_copy(
            table_hbm.at[idx_v.at[pl.ds(s * BLK, BLK)]],
            rows_v.at[b],
            gsem,
        )

    def store(s, b):
        return pltpu.make_async_copy(
            rows_v.at[b],
            out_hbm.at[pl.ds(base_w + s * BLK, BLK)],
            ssem,
        )

    def group(g, carry):
        for b in range(RING):
            s = g * RING + b

            @pl.when(s >= RING)
            def _wait_buffer_free():
                store(0, b).wait()

            gather(s, b).start()

            @pl.when(s >= 1)
            def _drain_prev_and_store():
                gather(0, 1 - b).wait()
                store(s - 1, 1 - b).start()

        return carry

    lax.fori_loop(0, SLOTS // RING, group, 0)

    b_last = (SLOTS - 1) % RING
    gather(0, b_last).wait()
    store(SLOTS - 1, b_last).start()
    store(0, 0).wait()
    store(0, 1).wait()


def kernel(modality_ids, modality_embedding):
    ids = lax.bitcast_convert_type(
        modality_ids.astype(jnp.int8).reshape(NW, PER_W, G), jnp.int32
    )
    t = modality_embedding
    t2 = jnp.concatenate(
        [jnp.repeat(t, 3, axis=0), jnp.tile(t, (3, 1))], axis=1
    )
    t4 = jnp.concatenate(
        [jnp.repeat(t2, 9, axis=0), jnp.tile(t2, (9, 1))], axis=1
    )
    t4 = jnp.tile(t4, (NW, 1))

    mesh = plsc.VectorSubcoreMesh(core_axis_name="c", subcore_axis_name="s")
    run = functools.partial(
        pl.kernel,
        mesh=mesh,
        compiler_params=pltpu.CompilerParams(use_tc_tiling_on_sc=False),
        out_type=jax.ShapeDtypeStruct((NGRP, ROWW), jnp.float32),
        scratch_types=[
            pltpu.VMEM((PER_W,), jnp.int32),
            pltpu.VMEM((PER_W,), jnp.int32),
            pltpu.VMEM((RING, BLK, ROWW), jnp.float32),
            pltpu.SemaphoreType.DMA,
            pltpu.SemaphoreType.DMA,
        ],
    )(_sc_body)
    out = run(ids, t4)
    return out.reshape(modality_ids.shape + (EMBED,))


# speedup vs baseline: 1.4089x; 1.4089x over previous
import functools

import jax
import jax.numpy as jnp
from jax import lax
from jax.experimental import pallas as pl
from jax.experimental.pallas import tpu as pltpu
from jax.experimental.pallas import tpu_sc as plsc

NI = 4096
NJ = 200
EMBED = 64
NC, NS = 2, 16
NW = NC * NS
DB, DS = 8, 8
IB, IL = NI // 128, 128
SLABS = NJ * DB
PER_W = SLABS // NW
RING = 2
L = 16

_GATHER_DNUMS = lax.GatherDimensionNumbers(
    offset_dims=(), collapsed_slice_dims=(0,), start_index_map=(0,)
)


def _sc_body(idst_hbm, tablet_hbm, out_hbm, ids_v, table_v, buf_v, ssem):
    wid = lax.axis_index("s") * NC + lax.axis_index("c")
    base_w = wid * PER_W

    pltpu.sync_copy(tablet_hbm, table_v)

    def store(j, db, b):
        return pltpu.make_async_copy(
            buf_v.at[b],
            out_hbm.at[j, db],
            ssem,
        )

    def slab(t, b):
        s = base_w + t
        j = s // DB
        db = s % DB
        pltpu.sync_copy(idst_hbm.at[j], ids_v)

        def ib_body(ib, carry):
            for g in range(IL // L):
                ids_g = ids_v[pl.ds(ib * IL + g * L, L)]
                for ds in range(DS):
                    trow = table_v[db * DS + ds, :]
                    v = lax.gather(
                        trow,
                        ids_g[:, None],
                        _GATHER_DNUMS,
                        (1,),
                        mode=lax.GatherScatterMode.PROMISE_IN_BOUNDS,
                    )
                    buf_v[b, ib, ds, pl.ds(g * L, L)] = v
            return carry

        lax.fori_loop(0, IB, ib_body, 0)
        return j, db

    def group(gidx, carry):
        for b in range(RING):
            t = gidx * RING + b

            @pl.when(t >= RING)
            def _wait_buffer_free():
                store(0, 0, b).wait()

            j, db = slab(t, b)
            store(j, db, b).start()

        return carry

    lax.fori_loop(0, PER_W // RING, group, 0)
    store(0, 0, 0).wait()
    store(0, 0, 1).wait()


def kernel(modality_ids, modality_embedding):
    idst = modality_ids.astype(jnp.int32).T
    tablet = jnp.zeros((EMBED, L), jnp.float32)
    tablet = tablet.at[:, :3].set(modality_embedding.T)

    mesh = plsc.VectorSubcoreMesh(core_axis_name="c", subcore_axis_name="s")
    run = functools.partial(
        pl.kernel,
        mesh=mesh,
        out_type=jax.ShapeDtypeStruct((NJ, DB, IB, DS, IL), jnp.float32),
        scratch_types=[
            pltpu.VMEM((NI,), jnp.int32),
            pltpu.VMEM((EMBED, L), jnp.float32),
            pltpu.VMEM((RING, IB, DS, IL), jnp.float32),
            pltpu.SemaphoreType.DMA,
        ],
    )(_sc_body)
    out5 = run(idst, tablet)
    return out5.transpose(2, 4, 0, 1, 3).reshape(NI, NJ, EMBED)


# speedup vs baseline: 3.9155x; 2.7791x over previous
import functools

import jax
import jax.numpy as jnp
from jax import lax
from jax.experimental import pallas as pl
from jax.experimental.pallas import tpu as pltpu
from jax.experimental.pallas import tpu_sc as plsc

NI = 4096
NJ = 200
EMBED = 64
NC, NS = 2, 16
NW = NC * NS
DB, DS = 8, 8
IB, IL = NI // 128, 128
SLABS = NJ * DB
PER_W = SLABS // NW
RING = 2
L = 16


def _sc_body(idst_hbm, tables_hbm, out_hbm, ids_v, table_v, buf_v, ssem):
    wid = lax.axis_index("s") * NC + lax.axis_index("c")
    base_w = wid * PER_W

    pltpu.sync_copy(tables_hbm, table_v)

    def store(j, db, b):
        return pltpu.make_async_copy(
            buf_v.at[b],
            out_hbm.at[j, db],
            ssem,
        )

    def slab(t, b):
        s = base_w + t
        j = s // DB
        db = s % DB
        pltpu.sync_copy(idst_hbm.at[j], ids_v)

        t0s = [table_v[0, db * DS + ds, :] for ds in range(DS)]
        t1s = [table_v[1, db * DS + ds, :] for ds in range(DS)]
        t2s = [table_v[2, db * DS + ds, :] for ds in range(DS)]

        def ib_body(ib, carry):
            for g in range(IL // L):
                ids_g = ids_v[pl.ds(ib * IL + g * L, L)]
                m1 = ids_g == 1
                m2 = ids_g == 2
                for ds in range(DS):
                    v = jnp.where(m1, t1s[ds], jnp.where(m2, t2s[ds], t0s[ds]))
                    buf_v[b, ib, ds, pl.ds(g * L, L)] = v
            return carry

        lax.fori_loop(0, IB, ib_body, 0)
        return j, db

    def group(gidx, carry):
        for b in range(RING):
            t = gidx * RING + b

            @pl.when(t >= RING)
            def _wait_buffer_free():
                store(0, 0, b).wait()

            j, db = slab(t, b)
            store(j, db, b).start()

        return carry

    lax.fori_loop(0, PER_W // RING, group, 0)
    store(0, 0, 0).wait()
    store(0, 0, 1).wait()


def kernel(modality_ids, modality_embedding):
    idst = modality_ids.astype(jnp.int32).T
    tables = jnp.tile(modality_embedding[:, :, None], (1, 1, L))

    mesh = plsc.VectorSubcoreMesh(core_axis_name="c", subcore_axis_name="s")
    run = functools.partial(
        pl.kernel,
        mesh=mesh,
        out_type=jax.ShapeDtypeStruct((NJ, DB, IB, DS, IL), jnp.float32),
        scratch_types=[
            pltpu.VMEM((NI,), jnp.int32),
            pltpu.VMEM((3, EMBED, L), jnp.float32),
            pltpu.VMEM((RING, IB, DS, IL), jnp.float32),
            pltpu.SemaphoreType.DMA,
        ],
    )(_sc_body)
    out5 = run(idst, tables)
    return out5.transpose(2, 4, 0, 1, 3).reshape(NI, NJ, EMBED)


# speedup vs baseline: 7.9033x; 2.0184x over previous
import functools

import jax
import jax.numpy as jnp
from jax import lax
from jax.experimental import pallas as pl
from jax.experimental.pallas import tpu as pltpu
from jax.experimental.pallas import tpu_sc as plsc

NI = 4096
NJ = 200
EMBED = 64
NC, NS = 2, 16
NW = NC * NS
DB, DS = 8, 8
IB, IL = NI // 128, 128
SLABS = NJ * DB
PER_W = SLABS // NW
RING = 2
L = 16


def _sc_body(idst_hbm, tables_hbm, out_hbm, ids_v, table_v, buf_v, ssem):
    wid = lax.axis_index("s") * NC + lax.axis_index("c")
    base_w = wid * PER_W

    pltpu.sync_copy(tables_hbm, table_v)

    def store(j, db, b):
        return pltpu.make_async_copy(
            buf_v.at[b],
            out_hbm.at[j, db],
            ssem,
        )

    def slab(t, b):
        s = base_w + t
        j = s // DB
        db = s % DB

        @pl.when(jnp.logical_or(t == 0, db == 0))
        def _stage_ids():
            pltpu.sync_copy(idst_hbm.at[j], ids_v)

        t0s = [table_v[0, db * DS + ds, :] for ds in range(DS)]
        t1s = [table_v[1, db * DS + ds, :] for ds in range(DS)]
        t2s = [table_v[2, db * DS + ds, :] for ds in range(DS)]

        @plsc.parallel_loop(0, IB, unroll=2)
        def ib_body(ib):
            for g in range(IL // L):
                ids_g = ids_v[pl.ds(ib * IL + g * L, L)]
                m1 = ids_g == 1
                m2 = ids_g == 2
                for ds in range(DS):
                    v = jnp.where(m1, t1s[ds], jnp.where(m2, t2s[ds], t0s[ds]))
                    buf_v[b, ib, ds, pl.ds(g * L, L)] = v

        return j, db

    def group(gidx, carry):
        for b in range(RING):
            t = gidx * RING + b

            @pl.when(t >= RING)
            def _wait_buffer_free():
                store(0, 0, b).wait()

            j, db = slab(t, b)
            store(j, db, b).start()

        return carry

    lax.fori_loop(0, PER_W // RING, group, 0)
    store(0, 0, 0).wait()
    store(0, 0, 1).wait()


def kernel(modality_ids, modality_embedding):
    idst = modality_ids.astype(jnp.int32).T
    tables = jnp.tile(modality_embedding[:, :, None], (1, 1, L))

    mesh = plsc.VectorSubcoreMesh(core_axis_name="c", subcore_axis_name="s")
    run = functools.partial(
        pl.kernel,
        mesh=mesh,
        out_type=jax.ShapeDtypeStruct((NJ, DB, IB, DS, IL), jnp.float32),
        scratch_types=[
            pltpu.VMEM((NI,), jnp.int32),
            pltpu.VMEM((3, EMBED, L), jnp.float32),
            pltpu.VMEM((RING, IB, DS, IL), jnp.float32),
            pltpu.SemaphoreType.DMA,
        ],
    )(_sc_body)
    out5 = run(idst, tables)
    return out5.transpose(2, 4, 0, 1, 3).reshape(NI, NJ, EMBED)
